# baseline (speedup 1.0000x reference)
import functools

import jax
import jax.numpy as jnp
from jax import lax
from jax.experimental import pallas as pl
from jax.experimental.pallas import tpu as pltpu
from jax.experimental.pallas import tpu_sc as plsc

EPS = 1e-5
K = 16


def _bn(x, g, b):
    m = jnp.mean(x, axis=0)
    v = jnp.var(x, axis=0)
    return (x - m) / jnp.sqrt(v + EPS) * g + b


def _make_fps_kernel(n, rows, cols, ns):
    def _fps_kernel(px_ref, py_ref, pz_ref, pxs_ref, pys_ref, pzs_ref,
                    idx_ref):
        ii = jax.lax.broadcasted_iota(jnp.int32, (rows, cols), 0) * cols + \
            jax.lax.broadcasted_iota(jnp.int32, (rows, cols), 1)
        valid = ii < n
        px = px_ref[...]
        py = py_ref[...]
        pz = pz_ref[...]
        dists0 = jnp.where(valid, jnp.inf, -jnp.inf).astype(jnp.float32)
        idx_ref[0:1, :] = jnp.zeros((1, 1), jnp.int32)

        def body(i, carry):
            dists, last = carry
            lx = pxs_ref[last]
            ly = pys_ref[last]
            lz = pzs_ref[last]
            dx = px - lx
            dy = py - ly
            dz = pz - lz
            d = dx * dx + dy * dy + dz * dz
            dists = jnp.minimum(dists, d)
            m = jnp.max(dists)
            nxt = jnp.min(jnp.where(dists == m, ii, jnp.int32(rows * cols)))
            idx_ref[pl.ds(i, 1), :] = jnp.full((1, 1), nxt, jnp.int32)
            return dists, nxt

        jax.lax.fori_loop(1, ns, body, (dists0, jnp.int32(0)))

    return _fps_kernel


def _fps(pos, n_samples):
    n = pos.shape[0]
    cols = 128
    rows = (n + cols - 1) // cols
    npad = rows * cols
    posp = jnp.pad(pos, ((0, npad - n), (0, 0)))
    px = posp[:, 0].reshape(rows, cols)
    py = posp[:, 1].reshape(rows, cols)
    pz = posp[:, 2].reshape(rows, cols)
    vspec = pl.BlockSpec((rows, cols), lambda: (0, 0))
    sspec = pl.BlockSpec(memory_space=pltpu.SMEM)
    idx = pl.pallas_call(
        _make_fps_kernel(n, rows, cols, n_samples),
        in_specs=[vspec, vspec, vspec, sspec, sspec, sspec],
        out_shape=jax.ShapeDtypeStruct((n_samples, 1), jnp.int32),
    )(px, py, pz, posp[:, 0], posp[:, 1], posp[:, 2])
    return jnp.sort(idx[:, 0])


KNN_R = 512


def _make_knn_kernel(m, mpad, k):
    def _knn_kernel(f_ref, ft_ref, nbr_ref):
        i = pl.program_id(0)
        fb = f_ref[...]
        ft = ft_ref[...]
        sq_blk = jnp.sum(fb * fb, axis=1, keepdims=True)
        sq_all = jnp.sum(ft * ft, axis=0, keepdims=True)
        d = sq_blk + sq_all - 2.0 * jnp.dot(fb, ft,
                                            preferred_element_type=jnp.float32)
        col = jax.lax.broadcasted_iota(jnp.int32, (KNN_R, mpad), 1)
        row = i * KNN_R + jax.lax.broadcasted_iota(jnp.int32, (KNN_R, mpad), 0)
        d = jnp.where((col == row) | (col >= m), jnp.inf, d)
        cols = []
        for _ in range(k):
            mn = jnp.min(d, axis=1, keepdims=True)
            am = jnp.min(jnp.where(d == mn, col, jnp.int32(mpad)), axis=1,
                         keepdims=True)
            cols.append(am)
            d = jnp.where(col == am, jnp.inf, d)
        nbr_ref[...] = jnp.concatenate(cols, axis=1)

    return _knn_kernel


def _knn_nbr(f, k):
    m, c = f.shape
    mpad = (m + KNN_R - 1) // KNN_R * KNN_R
    fp = jnp.pad(f, ((0, mpad - m), (0, 0)))
    nbr = pl.pallas_call(
        _make_knn_kernel(m, mpad, k),
        grid=(mpad // KNN_R,),
        in_specs=[
            pl.BlockSpec((KNN_R, c), lambda i: (i, 0)),
            pl.BlockSpec((c, mpad), lambda i: (0, 0)),
        ],
        out_specs=pl.BlockSpec((KNN_R, k), lambda i: (i, 0)),
        out_shape=jax.ShapeDtypeStruct((mpad, k), jnp.int32),
    )(fp, fp.T)
    return nbr[:m]


def _max_pool_neighbor(x, edge_index, N):
    loop = jnp.arange(N, dtype=jnp.int32)
    src = jnp.concatenate([edge_index[0].astype(jnp.int32), loop])
    dst = jnp.concatenate([edge_index[1].astype(jnp.int32), loop])
    return jax.ops.segment_max(x[src], dst, num_segments=N)


def _mm_kernel(x_ref, w_ref, o_ref):
    o_ref[...] = jnp.dot(x_ref[...], w_ref[...],
                         preferred_element_type=jnp.float32)


def _pallas_matmul(x, w, blk=512):
    n, cin = x.shape
    cout = w.shape[1]
    n_pad = (n + blk - 1) // blk * blk
    xp = jnp.pad(x, ((0, n_pad - n), (0, 0)))
    out = pl.pallas_call(
        _mm_kernel,
        grid=(n_pad // blk,),
        in_specs=[
            pl.BlockSpec((blk, cin), lambda i: (i, 0)),
            pl.BlockSpec((cin, cout), lambda i: (0, 0)),
        ],
        out_specs=pl.BlockSpec((blk, cout), lambda i: (i, 0)),
        out_shape=jax.ShapeDtypeStruct((n_pad, cout), jnp.float32),
    )(xp, w)
    return out[:n]


def _sc_gather3(tab_a, tab_b, tab_p, idx):
    e_total = idx.shape[0]
    info = plsc.get_sparse_core_info()
    nw = info.num_cores * info.num_subcores
    per_w = e_total // nw
    ch = 40
    n_ch = per_w // ch
    assert per_w % ch == 0 and per_w % 8 == 0
    mesh = plsc.VectorSubcoreMesh(core_axis_name="c", subcore_axis_name="s")

    @functools.partial(
        pl.kernel, mesh=mesh,
        out_type=[
            jax.ShapeDtypeStruct((e_total, 256), jnp.float32),
            jax.ShapeDtypeStruct((e_total, 256), jnp.float32),
            jax.ShapeDtypeStruct((e_total, 128), jnp.float32),
        ],
        scratch_types=[
            pltpu.VMEM((ch,), jnp.int32),
            pltpu.VMEM((ch, 256), jnp.float32),
            pltpu.VMEM((ch, 256), jnp.float32),
            pltpu.VMEM((ch, 128), jnp.float32),
            pltpu.SemaphoreType.DMA,
            pltpu.SemaphoreType.DMA,
            pltpu.SemaphoreType.DMA,
        ],
    )
    def gk(ta_hbm, tb_hbm, tp_hbm, idx_hbm, oa_hbm, ob_hbm, op_hbm,
           idx_v, buf_a, buf_b, buf_p, sem_a, sem_b, sem_p):
        wid = lax.axis_index("s") * info.num_cores + lax.axis_index("c")
        base = wid * per_w

        def body(c, _):
            off = base + c * ch
            pltpu.sync_copy(idx_hbm.at[pl.ds(off, ch)], idx_v)
            ca = pltpu.async_copy(ta_hbm.at[idx_v], buf_a, sem_a)
            cb = pltpu.async_copy(tb_hbm.at[idx_v], buf_b, sem_b)
            cp = pltpu.async_copy(tp_hbm.at[idx_v], buf_p, sem_p)
            ca.wait()
            cb.wait()
            cp.wait()
            pltpu.sync_copy(buf_a, oa_hbm.at[pl.ds(off, ch)])
            pltpu.sync_copy(buf_b, ob_hbm.at[pl.ds(off, ch)])
            pltpu.sync_copy(buf_p, op_hbm.at[pl.ds(off, ch)])
            return 0

        lax.fori_loop(0, n_ch, body, 0)

    return gk(tab_a, tab_b, tab_p, idx)


NBM = 40


def _stats_update(i, s_ref, q_ref, val):
    ps = jnp.sum(val, axis=(0, 1), keepdims=False).reshape(1, -1)
    pq = jnp.sum(val * val, axis=(0, 1), keepdims=False).reshape(1, -1)

    @pl.when(i == 0)
    def _():
        s_ref[...] = ps
        q_ref[...] = pq

    @pl.when(i > 0)
    def _():
        s_ref[...] += ps
        q_ref[...] += pq


def _bn_from_stats(u, s, q, g, be, count):
    m = s / count
    v = q / count - m * m
    return (u - m) / jnp.sqrt(v + EPS) * g + be


def _edge_pm_kernel(pos_sg_ref, pos_s_ref, m1_ref, mx_ref, my_ref, mz_ref):
    i = pl.program_id(0)
    pd = jnp.broadcast_to(pos_s_ref[...][None, :, :], (32, NBM, 3))
    p = pd - pos_sg_ref[...]
    outs = [
        jnp.sum(p, axis=(0, 1)).reshape(1, 3),
        jnp.sum(p * p[:, :, 0:1], axis=(0, 1)).reshape(1, 3),
        jnp.sum(p * p[:, :, 1:2], axis=(0, 1)).reshape(1, 3),
        jnp.sum(p * p[:, :, 2:3], axis=(0, 1)).reshape(1, 3),
    ]
    for ref, val in zip((m1_ref, mx_ref, my_ref, mz_ref), outs):
        @pl.when(i == 0)
        def _(ref=ref, val=val):
            ref[...] = val

        @pl.when(i > 0)
        def _(ref=ref, val=val):
            ref[...] += val


def _edge_p2_kernel(pos_sg_ref, pos_s_ref, m1_ref, mx_ref, my_ref, mz_ref,
                    w1_ref, b1_ref, g_ref, be_ref, w2_ref, b2_ref,
                    u2_ref, s_ref, q_ref, *, count):
    i = pl.program_id(0)
    w1 = w1_ref[...]
    b1 = b1_ref[...]
    s1 = jnp.dot(m1_ref[...], w1, preferred_element_type=jnp.float32) \
        + count * b1
    tx = jnp.dot(mx_ref[...], w1, preferred_element_type=jnp.float32)
    ty = jnp.dot(my_ref[...], w1, preferred_element_type=jnp.float32)
    tz = jnp.dot(mz_ref[...], w1, preferred_element_type=jnp.float32)
    q1 = (w1[0:1, :] * tx + w1[1:2, :] * ty + w1[2:3, :] * tz
          + 2.0 * b1 * (s1 - count * b1) + count * b1 * b1)
    pd = jnp.broadcast_to(pos_s_ref[...][None, :, :], (32, NBM, 3))
    p = pd - pos_sg_ref[...]
    u1 = jnp.dot(p.reshape(32 * NBM, 3), w1,
                 preferred_element_type=jnp.float32) + b1
    h = _bn_from_stats(u1.reshape(32, NBM, 256), s1, q1, g_ref[...],
                       be_ref[...], count)
    h = jax.nn.relu(h)
    u2 = jnp.dot(h.reshape(32 * NBM, 256), w2_ref[...],
                 preferred_element_type=jnp.float32) + b2_ref[...]
    u2_ref[...] = u2.reshape(32, NBM, 256)
    _stats_update(i, s_ref, q_ref, u2_ref[...])


def _edge_mid_kernel(u_ref, s_in_ref, q_in_ref, g_ref, be_ref, w_ref, b_ref,
                     u2_ref, s_ref, q_ref, *, count):
    i = pl.program_id(0)
    h = _bn_from_stats(u_ref[...], s_in_ref[...], q_in_ref[...], g_ref[...],
                       be_ref[...], count)
    h = jax.nn.relu(h)
    u2 = jnp.dot(h.reshape(32 * NBM, 256), w_ref[...],
                 preferred_element_type=jnp.float32) + b_ref[...]
    u2_ref[...] = u2.reshape(32, NBM, 256)
    _stats_update(i, s_ref, q_ref, u2_ref[...])


def _edge_a1_kernel(u_ref, s_in_ref, q_in_ref, g_ref, be_ref, a_srcg_ref,
                    a_dst_ref, w_ref, b_ref, v1_ref, delta_ref, s_ref, q_ref,
                    *, count):
    i = pl.program_id(0)
    delta = _bn_from_stats(u_ref[...], s_in_ref[...], q_in_ref[...],
                           g_ref[...], be_ref[...], count)
    delta = jax.nn.relu(delta)
    delta_ref[...] = delta
    ad = jnp.broadcast_to(a_dst_ref[...][None, :, :], (32, NBM, 256))
    gacc = ad - a_srcg_ref[...] + delta
    v1 = jnp.dot(gacc.reshape(32 * NBM, 256), w_ref[...],
                 preferred_element_type=jnp.float32) + b_ref[...]
    v1_ref[...] = v1.reshape(32, NBM, 256)
    _stats_update(i, s_ref, q_ref, v1_ref[...])


def _edge_final_kernel(v2_ref, s_in_ref, q_in_ref, g_ref, be_ref, delta_ref,
                       xval_g_ref, w_ref, out_ref, *, count):
    a = _bn_from_stats(v2_ref[...], s_in_ref[...], q_in_ref[...], g_ref[...],
                       be_ref[...], count)
    a = jax.nn.relu(a)
    mx = jnp.max(a, axis=0, keepdims=True)
    e = jnp.exp(a - mx)
    ssum = jnp.sum(e, axis=0, keepdims=True)
    alpha = e / (ssum + 1e-16)
    msg = alpha * (xval_g_ref[...] + delta_ref[...]) * w_ref[...]
    out_ref[...] = jnp.sum(msg, axis=0)


def _edge_pipeline(pos_s, pos_sg, a_dst, a_srcg, xval_g, w_edge, W_p1, b_p1,
                   g_p1, be_p1, W_p2, b_p2, g_p2, be_p2, W_a1, b_a1, g_a1,
                   be_a1, W_a2, b_a2, g_a2, be_a2):
    M = pos_s.shape[0]
    E = float(32 * M)
    nblk = M // NBM
    grid = (nblk,)
    C = 256

    def espec(c):
        return pl.BlockSpec((32, NBM, c), lambda i: (0, i, 0))

    def nspec(c):
        return pl.BlockSpec((NBM, c), lambda i: (i, 0))

    def wspec(r, c):
        return pl.BlockSpec((r, c), lambda i: (0, 0))

    stat_spec = pl.BlockSpec((1, C), lambda i: (0, 0))
    stat_shape = jax.ShapeDtypeStruct((1, C), jnp.float32)
    eshape = jax.ShapeDtypeStruct((32, M, C), jnp.float32)

    mspec = pl.BlockSpec((1, 3), lambda i: (0, 0))
    mshape = jax.ShapeDtypeStruct((1, 3), jnp.float32)
    m1, mx, my, mz = pl.pallas_call(
        _edge_pm_kernel,
        grid=grid,
        in_specs=[espec(3), nspec(3)],
        out_specs=[mspec, mspec, mspec, mspec],
        out_shape=[mshape, mshape, mshape, mshape],
    )(pos_sg, pos_s)

    u2, s2, q2 = pl.pallas_call(
        functools.partial(_edge_p2_kernel, count=E),
        grid=grid,
        in_specs=[espec(3), nspec(3), mspec, mspec, mspec, mspec,
                  wspec(3, C), wspec(1, C), wspec(1, C), wspec(1, C),
                  wspec(C, C), wspec(1, C)],
        out_specs=[espec(C), stat_spec, stat_spec],
        out_shape=[eshape, stat_shape, stat_shape],
    )(pos_sg, pos_s, m1, mx, my, mz, W_p1, b_p1.reshape(1, C),
      g_p1.reshape(1, C), be_p1.reshape(1, C), W_p2, b_p2.reshape(1, C))

    v1, delta, s3, q3 = pl.pallas_call(
        functools.partial(_edge_a1_kernel, count=E),
        grid=grid,
        in_specs=[espec(C), stat_spec, stat_spec, wspec(1, C), wspec(1, C),
                  espec(C), nspec(C), wspec(C, C), wspec(1, C)],
        out_specs=[espec(C), espec(C), stat_spec, stat_spec],
        out_shape=[eshape, eshape, stat_shape, stat_shape],
    )(u2, s2, q2, g_p2.reshape(1, C), be_p2.reshape(1, C), a_srcg, a_dst,
      W_a1, b_a1.reshape(1, C))

    v2, s4, q4 = pl.pallas_call(
        functools.partial(_edge_mid_kernel, count=E),
        grid=grid,
        in_specs=[espec(C), stat_spec, stat_spec, wspec(1, C), wspec(1, C),
                  wspec(C, C), wspec(1, C)],
        out_specs=[espec(C), stat_spec, stat_spec],
        out_shape=[eshape, stat_shape, stat_shape],
    )(v1, s3, q3, g_a1.reshape(1, C), be_a1.reshape(1, C), W_a2,
      b_a2.reshape(1, C))

    out = pl.pallas_call(
        functools.partial(_edge_final_kernel, count=E),
        grid=grid,
        in_specs=[espec(C), stat_spec, stat_spec, wspec(1, C), wspec(1, C),
                  espec(C), espec(C),
                  pl.BlockSpec((32, NBM, 1), lambda i: (0, i, 0))],
        out_specs=nspec(C),
        out_shape=jax.ShapeDtypeStruct((M, C), jnp.float32),
    )(v2, s4, q4, g_a2.reshape(1, C), be_a2.reshape(1, C), delta, xval_g,
      w_edge[:, :, None])
    return out


def kernel(x, pos, batch, y, edge_index, W_down, b_down, g_down, be_down,
           W_g, b_g, g_g, be_g, t, W_lin, W_src, W_dst, W_p1, b_p1, g_p1,
           be_p1, W_p2, b_p2, g_p2, be_p2, W_a1, b_a1, g_a1, be_a1, W_a2,
           b_a2, g_a2, be_a2, W_up, b_up):
    N = x.shape[0]
    h = jax.nn.relu(_bn(_pallas_matmul(x, W_down) + b_down, g_down, be_down))
    h = _max_pool_neighbor(h, edge_index, N)
    idx = _fps(pos, N // 2)
    x_s = h[idx]
    pos_s = pos[idx]
    M = x_s.shape[0]

    emb = jax.nn.relu(_bn(x_s @ W_g + b_g, g_g, be_g))
    soft_nbr = _knn_nbr(emb, K)
    hard_nbr = _knn_nbr(pos_s, K)
    noise = jax.random.uniform(jax.random.key(42), emb.shape,
                               dtype=emb.dtype) * 1e-4
    embn = emb + noise

    src = jnp.concatenate([soft_nbr.T, hard_nbr.T], axis=0)

    diff = embn[soft_nbr.T] - embn[None, :, :]
    dist = jnp.sqrt(jnp.sum(diff * diff, axis=2) + 1e-12)
    p = jnp.exp(-t[0] * dist)
    w_edge = jnp.concatenate([p, jnp.ones((16, M), p.dtype)], axis=0)

    xv3 = _pallas_matmul(x_s, jnp.concatenate([W_lin, W_src, W_dst], axis=1))
    x_val = xv3[:, :256]
    a_src = xv3[:, 256:512]
    a_dst = xv3[:, 512:]

    pos_pad = jnp.pad(pos_s, ((0, 0), (0, 125)))
    ag_f, xg_f, pg_f = _sc_gather3(a_src, x_val, pos_pad, src.reshape(-1))
    a_srcg = ag_f.reshape(32, M, 256)
    xval_g = xg_f.reshape(32, M, 256)
    pos_sg = pg_f.reshape(32, M, 128)[:, :, :3]

    out = _edge_pipeline(pos_s, pos_sg, a_dst, a_srcg, xval_g, w_edge,
                         W_p1, b_p1, g_p1, be_p1, W_p2, b_p2, g_p2, be_p2,
                         W_a1, b_a1, g_a1, be_a1, W_a2, b_a2, g_a2, be_a2)
    out = _pallas_matmul(out, W_up) + b_up
    return out + x_s


# speedup vs baseline: 1.0362x; 1.0362x over previous
import functools

import jax
import jax.numpy as jnp
from jax import lax
from jax.experimental import pallas as pl
from jax.experimental.pallas import tpu as pltpu
from jax.experimental.pallas import tpu_sc as plsc

EPS = 1e-5
K = 16


def _bn(x, g, b):
    m = jnp.mean(x, axis=0)
    v = jnp.var(x, axis=0)
    return (x - m) / jnp.sqrt(v + EPS) * g + b


def _make_fps_kernel(n, rows, cols, ns):
    def _fps_kernel(px_ref, py_ref, pz_ref, pxs_ref, pys_ref, pzs_ref,
                    idx_ref):
        ii = jax.lax.broadcasted_iota(jnp.int32, (rows, cols), 0) * cols + \
            jax.lax.broadcasted_iota(jnp.int32, (rows, cols), 1)
        valid = ii < n
        px = px_ref[...]
        py = py_ref[...]
        pz = pz_ref[...]
        dists0 = jnp.where(valid, jnp.inf, -jnp.inf).astype(jnp.float32)
        idx_ref[0:1, :] = jnp.zeros((1, 1), jnp.int32)

        def body(i, carry):
            dists, last = carry
            lx = pxs_ref[last]
            ly = pys_ref[last]
            lz = pzs_ref[last]
            dx = px - lx
            dy = py - ly
            dz = pz - lz
            d = dx * dx + dy * dy + dz * dz
            dists = jnp.minimum(dists, d)
            m = jnp.max(dists)
            nxt = jnp.min(jnp.where(dists == m, ii, jnp.int32(rows * cols)))
            idx_ref[pl.ds(i, 1), :] = jnp.full((1, 1), nxt, jnp.int32)
            return dists, nxt

        jax.lax.fori_loop(1, ns, body, (dists0, jnp.int32(0)))

    return _fps_kernel


def _fps(pos, n_samples):
    n = pos.shape[0]
    cols = 128
    rows = (n + cols - 1) // cols
    npad = rows * cols
    posp = jnp.pad(pos, ((0, npad - n), (0, 0)))
    px = posp[:, 0].reshape(rows, cols)
    py = posp[:, 1].reshape(rows, cols)
    pz = posp[:, 2].reshape(rows, cols)
    vspec = pl.BlockSpec((rows, cols), lambda: (0, 0))
    sspec = pl.BlockSpec(memory_space=pltpu.SMEM)
    idx = pl.pallas_call(
        _make_fps_kernel(n, rows, cols, n_samples),
        in_specs=[vspec, vspec, vspec, sspec, sspec, sspec],
        out_shape=jax.ShapeDtypeStruct((n_samples, 1), jnp.int32),
    )(px, py, pz, posp[:, 0], posp[:, 1], posp[:, 2])
    return jnp.sort(idx[:, 0])


KNN_R = 256


def _make_knn_kernel(m, mpad, k):
    def _knn_kernel(f_ref, ft_ref, nbr_ref):
        i = pl.program_id(0)
        fb = f_ref[...]
        ft = ft_ref[...]
        sq_blk = jnp.sum(fb * fb, axis=1, keepdims=True)
        sq_all = jnp.sum(ft * ft, axis=0, keepdims=True)
        d = sq_blk + sq_all - 2.0 * jnp.dot(fb, ft,
                                            preferred_element_type=jnp.float32)
        col = jax.lax.broadcasted_iota(jnp.int32, (KNN_R, mpad), 1)
        row = i * KNN_R + jax.lax.broadcasted_iota(jnp.int32, (KNN_R, mpad), 0)
        d = jnp.where((col == row) | (col >= m), jnp.inf, d)
        cols = []
        for _ in range(k):
            mn = jnp.min(d, axis=1, keepdims=True)
            am = jnp.min(jnp.where(d == mn, col, jnp.int32(mpad)), axis=1,
                         keepdims=True)
            cols.append(am)
            d = jnp.where(col == am, jnp.inf, d)
        nbr_ref[...] = jnp.concatenate(cols, axis=1)

    return _knn_kernel


def _knn_nbr(f, k):
    m, c = f.shape
    mpad = (m + KNN_R - 1) // KNN_R * KNN_R
    fp = jnp.pad(f, ((0, mpad - m), (0, 0)))
    nbr = pl.pallas_call(
        _make_knn_kernel(m, mpad, k),
        grid=(mpad // KNN_R,),
        in_specs=[
            pl.BlockSpec((KNN_R, c), lambda i: (i, 0)),
            pl.BlockSpec((c, mpad), lambda i: (0, 0)),
        ],
        out_specs=pl.BlockSpec((KNN_R, k), lambda i: (i, 0)),
        out_shape=jax.ShapeDtypeStruct((mpad, k), jnp.int32),
    )(fp, fp.T)
    return nbr[:m]


def _max_pool_neighbor(x, edge_index, N):
    loop = jnp.arange(N, dtype=jnp.int32)
    src = jnp.concatenate([edge_index[0].astype(jnp.int32), loop])
    dst = jnp.concatenate([edge_index[1].astype(jnp.int32), loop])
    return jax.ops.segment_max(x[src], dst, num_segments=N)


def _mm_kernel(x_ref, w_ref, o_ref):
    o_ref[...] = jnp.dot(x_ref[...], w_ref[...],
                         preferred_element_type=jnp.float32)


def _pallas_matmul(x, w, blk=512):
    n, cin = x.shape
    cout = w.shape[1]
    n_pad = (n + blk - 1) // blk * blk
    xp = jnp.pad(x, ((0, n_pad - n), (0, 0)))
    out = pl.pallas_call(
        _mm_kernel,
        grid=(n_pad // blk,),
        in_specs=[
            pl.BlockSpec((blk, cin), lambda i: (i, 0)),
            pl.BlockSpec((cin, cout), lambda i: (0, 0)),
        ],
        out_specs=pl.BlockSpec((blk, cout), lambda i: (i, 0)),
        out_shape=jax.ShapeDtypeStruct((n_pad, cout), jnp.float32),
    )(xp, w)
    return out[:n]


def _sc_gather3(tab_a, tab_b, tab_p, idx):
    e_total = idx.shape[0]
    info = plsc.get_sparse_core_info()
    nw = info.num_cores * info.num_subcores
    per_w = e_total // nw
    ch = 40
    n_ch = per_w // ch
    assert per_w % ch == 0 and per_w % 8 == 0
    mesh = plsc.VectorSubcoreMesh(core_axis_name="c", subcore_axis_name="s")

    @functools.partial(
        pl.kernel, mesh=mesh,
        out_type=[
            jax.ShapeDtypeStruct((e_total, 256), jnp.float32),
            jax.ShapeDtypeStruct((e_total, 256), jnp.float32),
            jax.ShapeDtypeStruct((e_total, 128), jnp.float32),
        ],
        scratch_types=[
            pltpu.VMEM((ch,), jnp.int32),
            pltpu.VMEM((ch, 256), jnp.float32),
            pltpu.VMEM((ch, 256), jnp.float32),
            pltpu.VMEM((ch, 128), jnp.float32),
            pltpu.SemaphoreType.DMA,
            pltpu.SemaphoreType.DMA,
            pltpu.SemaphoreType.DMA,
        ],
    )
    def gk(ta_hbm, tb_hbm, tp_hbm, idx_hbm, oa_hbm, ob_hbm, op_hbm,
           idx_v, buf_a, buf_b, buf_p, sem_a, sem_b, sem_p):
        wid = lax.axis_index("s") * info.num_cores + lax.axis_index("c")
        base = wid * per_w

        def body(c, _):
            off = base + c * ch
            pltpu.sync_copy(idx_hbm.at[pl.ds(off, ch)], idx_v)
            ca = pltpu.async_copy(ta_hbm.at[idx_v], buf_a, sem_a)
            cb = pltpu.async_copy(tb_hbm.at[idx_v], buf_b, sem_b)
            cp = pltpu.async_copy(tp_hbm.at[idx_v], buf_p, sem_p)
            ca.wait()
            cb.wait()
            cp.wait()
            pltpu.sync_copy(buf_a, oa_hbm.at[pl.ds(off, ch)])
            pltpu.sync_copy(buf_b, ob_hbm.at[pl.ds(off, ch)])
            pltpu.sync_copy(buf_p, op_hbm.at[pl.ds(off, ch)])
            return 0

        lax.fori_loop(0, n_ch, body, 0)

    return gk(tab_a, tab_b, tab_p, idx)


NBM = 40


def _stats_update(i, s_ref, q_ref, val):
    ps = jnp.sum(val, axis=(0, 1), keepdims=False).reshape(1, -1)
    pq = jnp.sum(val * val, axis=(0, 1), keepdims=False).reshape(1, -1)

    @pl.when(i == 0)
    def _():
        s_ref[...] = ps
        q_ref[...] = pq

    @pl.when(i > 0)
    def _():
        s_ref[...] += ps
        q_ref[...] += pq


def _bn_from_stats(u, s, q, g, be, count):
    m = s / count
    v = q / count - m * m
    return (u - m) / jnp.sqrt(v + EPS) * g + be


def _edge_pm_kernel(pos_sg_ref, pos_s_ref, m1_ref, mx_ref, my_ref, mz_ref):
    i = pl.program_id(0)
    pd = jnp.broadcast_to(pos_s_ref[...][None, :, :], (32, NBM, 3))
    p = pd - pos_sg_ref[...]
    outs = [
        jnp.sum(p, axis=(0, 1)).reshape(1, 3),
        jnp.sum(p * p[:, :, 0:1], axis=(0, 1)).reshape(1, 3),
        jnp.sum(p * p[:, :, 1:2], axis=(0, 1)).reshape(1, 3),
        jnp.sum(p * p[:, :, 2:3], axis=(0, 1)).reshape(1, 3),
    ]
    for ref, val in zip((m1_ref, mx_ref, my_ref, mz_ref), outs):
        @pl.when(i == 0)
        def _(ref=ref, val=val):
            ref[...] = val

        @pl.when(i > 0)
        def _(ref=ref, val=val):
            ref[...] += val


def _edge_p2_kernel(pos_sg_ref, pos_s_ref, m1_ref, mx_ref, my_ref, mz_ref,
                    w1_ref, b1_ref, g_ref, be_ref, w2_ref, b2_ref,
                    u2_ref, s_ref, q_ref, *, count):
    i = pl.program_id(0)
    w1 = w1_ref[...]
    b1 = b1_ref[...]
    s1 = jnp.dot(m1_ref[...], w1, preferred_element_type=jnp.float32) \
        + count * b1
    tx = jnp.dot(mx_ref[...], w1, preferred_element_type=jnp.float32)
    ty = jnp.dot(my_ref[...], w1, preferred_element_type=jnp.float32)
    tz = jnp.dot(mz_ref[...], w1, preferred_element_type=jnp.float32)
    q1 = (w1[0:1, :] * tx + w1[1:2, :] * ty + w1[2:3, :] * tz
          + 2.0 * b1 * (s1 - count * b1) + count * b1 * b1)
    pd = jnp.broadcast_to(pos_s_ref[...][None, :, :], (32, NBM, 3))
    p = pd - pos_sg_ref[...]
    u1 = jnp.dot(p.reshape(32 * NBM, 3), w1,
                 preferred_element_type=jnp.float32) + b1
    h = _bn_from_stats(u1.reshape(32, NBM, 256), s1, q1, g_ref[...],
                       be_ref[...], count)
    h = jax.nn.relu(h)
    u2 = jnp.dot(h.reshape(32 * NBM, 256), w2_ref[...],
                 preferred_element_type=jnp.float32) + b2_ref[...]
    u2_ref[...] = u2.reshape(32, NBM, 256)
    _stats_update(i, s_ref, q_ref, u2_ref[...])


def _edge_mid_kernel(u_ref, s_in_ref, q_in_ref, g_ref, be_ref, w_ref, b_ref,
                     u2_ref, s_ref, q_ref, *, count):
    i = pl.program_id(0)
    h = _bn_from_stats(u_ref[...], s_in_ref[...], q_in_ref[...], g_ref[...],
                       be_ref[...], count)
    h = jax.nn.relu(h)
    u2 = jnp.dot(h.reshape(32 * NBM, 256), w_ref[...],
                 preferred_element_type=jnp.float32) + b_ref[...]
    u2_ref[...] = u2.reshape(32, NBM, 256)
    _stats_update(i, s_ref, q_ref, u2_ref[...])


def _edge_a1_kernel(u_ref, s_in_ref, q_in_ref, g_ref, be_ref, a_srcg_ref,
                    a_dst_ref, w_ref, b_ref, v1_ref, delta_ref, s_ref, q_ref,
                    *, count):
    i = pl.program_id(0)
    delta = _bn_from_stats(u_ref[...], s_in_ref[...], q_in_ref[...],
                           g_ref[...], be_ref[...], count)
    delta = jax.nn.relu(delta)
    delta_ref[...] = delta
    ad = jnp.broadcast_to(a_dst_ref[...][None, :, :], (32, NBM, 256))
    gacc = ad - a_srcg_ref[...] + delta
    v1 = jnp.dot(gacc.reshape(32 * NBM, 256), w_ref[...],
                 preferred_element_type=jnp.float32) + b_ref[...]
    v1_ref[...] = v1.reshape(32, NBM, 256)
    _stats_update(i, s_ref, q_ref, v1_ref[...])


def _edge_final_kernel(v2_ref, s_in_ref, q_in_ref, g_ref, be_ref, delta_ref,
                       xval_g_ref, w_ref, out_ref, *, count):
    a = _bn_from_stats(v2_ref[...], s_in_ref[...], q_in_ref[...], g_ref[...],
                       be_ref[...], count)
    a = jax.nn.relu(a)
    mx = jnp.max(a, axis=0, keepdims=True)
    e = jnp.exp(a - mx)
    ssum = jnp.sum(e, axis=0, keepdims=True)
    alpha = e / (ssum + 1e-16)
    msg = alpha * (xval_g_ref[...] + delta_ref[...]) * w_ref[...]
    out_ref[...] = jnp.sum(msg, axis=0)


def _edge_pipeline(pos_s, pos_sg, a_dst, a_srcg, xval_g, w_edge, W_p1, b_p1,
                   g_p1, be_p1, W_p2, b_p2, g_p2, be_p2, W_a1, b_a1, g_a1,
                   be_a1, W_a2, b_a2, g_a2, be_a2):
    M = pos_s.shape[0]
    E = float(32 * M)
    nblk = M // NBM
    grid = (nblk,)
    C = 256

    def espec(c):
        return pl.BlockSpec((32, NBM, c), lambda i: (0, i, 0))

    def nspec(c):
        return pl.BlockSpec((NBM, c), lambda i: (i, 0))

    def wspec(r, c):
        return pl.BlockSpec((r, c), lambda i: (0, 0))

    stat_spec = pl.BlockSpec((1, C), lambda i: (0, 0))
    stat_shape = jax.ShapeDtypeStruct((1, C), jnp.float32)
    eshape = jax.ShapeDtypeStruct((32, M, C), jnp.float32)

    mspec = pl.BlockSpec((1, 3), lambda i: (0, 0))
    mshape = jax.ShapeDtypeStruct((1, 3), jnp.float32)
    m1, mx, my, mz = pl.pallas_call(
        _edge_pm_kernel,
        grid=grid,
        in_specs=[espec(3), nspec(3)],
        out_specs=[mspec, mspec, mspec, mspec],
        out_shape=[mshape, mshape, mshape, mshape],
    )(pos_sg, pos_s)

    u2, s2, q2 = pl.pallas_call(
        functools.partial(_edge_p2_kernel, count=E),
        grid=grid,
        in_specs=[espec(3), nspec(3), mspec, mspec, mspec, mspec,
                  wspec(3, C), wspec(1, C), wspec(1, C), wspec(1, C),
                  wspec(C, C), wspec(1, C)],
        out_specs=[espec(C), stat_spec, stat_spec],
        out_shape=[eshape, stat_shape, stat_shape],
    )(pos_sg, pos_s, m1, mx, my, mz, W_p1, b_p1.reshape(1, C),
      g_p1.reshape(1, C), be_p1.reshape(1, C), W_p2, b_p2.reshape(1, C))

    v1, delta, s3, q3 = pl.pallas_call(
        functools.partial(_edge_a1_kernel, count=E),
        grid=grid,
        in_specs=[espec(C), stat_spec, stat_spec, wspec(1, C), wspec(1, C),
                  espec(C), nspec(C), wspec(C, C), wspec(1, C)],
        out_specs=[espec(C), espec(C), stat_spec, stat_spec],
        out_shape=[eshape, eshape, stat_shape, stat_shape],
    )(u2, s2, q2, g_p2.reshape(1, C), be_p2.reshape(1, C), a_srcg, a_dst,
      W_a1, b_a1.reshape(1, C))

    v2, s4, q4 = pl.pallas_call(
        functools.partial(_edge_mid_kernel, count=E),
        grid=grid,
        in_specs=[espec(C), stat_spec, stat_spec, wspec(1, C), wspec(1, C),
                  wspec(C, C), wspec(1, C)],
        out_specs=[espec(C), stat_spec, stat_spec],
        out_shape=[eshape, stat_shape, stat_shape],
    )(v1, s3, q3, g_a1.reshape(1, C), be_a1.reshape(1, C), W_a2,
      b_a2.reshape(1, C))

    out = pl.pallas_call(
        functools.partial(_edge_final_kernel, count=E),
        grid=grid,
        in_specs=[espec(C), stat_spec, stat_spec, wspec(1, C), wspec(1, C),
                  espec(C), espec(C),
                  pl.BlockSpec((32, NBM, 1), lambda i: (0, i, 0))],
        out_specs=nspec(C),
        out_shape=jax.ShapeDtypeStruct((M, C), jnp.float32),
    )(v2, s4, q4, g_a2.reshape(1, C), be_a2.reshape(1, C), delta, xval_g,
      w_edge[:, :, None])
    return out


def kernel(x, pos, batch, y, edge_index, W_down, b_down, g_down, be_down,
           W_g, b_g, g_g, be_g, t, W_lin, W_src, W_dst, W_p1, b_p1, g_p1,
           be_p1, W_p2, b_p2, g_p2, be_p2, W_a1, b_a1, g_a1, be_a1, W_a2,
           b_a2, g_a2, be_a2, W_up, b_up):
    N = x.shape[0]
    h = jax.nn.relu(_bn(_pallas_matmul(x, W_down) + b_down, g_down, be_down))
    h = _max_pool_neighbor(h, edge_index, N)
    idx = _fps(pos, N // 2)
    x_s = h[idx]
    pos_s = pos[idx]
    M = x_s.shape[0]

    emb = jax.nn.relu(_bn(x_s @ W_g + b_g, g_g, be_g))
    soft_nbr = _knn_nbr(emb, K)
    hard_nbr = _knn_nbr(pos_s, K)
    noise = jax.random.uniform(jax.random.key(42), emb.shape,
                               dtype=emb.dtype) * 1e-4
    embn = emb + noise

    src = jnp.concatenate([soft_nbr.T, hard_nbr.T], axis=0)

    diff = embn[soft_nbr.T] - embn[None, :, :]
    dist = jnp.sqrt(jnp.sum(diff * diff, axis=2) + 1e-12)
    p = jnp.exp(-t[0] * dist)
    w_edge = jnp.concatenate([p, jnp.ones((16, M), p.dtype)], axis=0)

    xv3 = _pallas_matmul(x_s, jnp.concatenate([W_lin, W_src, W_dst], axis=1))
    x_val = xv3[:, :256]
    a_src = xv3[:, 256:512]
    a_dst = xv3[:, 512:]

    pos_pad = jnp.pad(pos_s, ((0, 0), (0, 125)))
    ag_f, xg_f, pg_f = _sc_gather3(a_src, x_val, pos_pad, src.reshape(-1))
    a_srcg = ag_f.reshape(32, M, 256)
    xval_g = xg_f.reshape(32, M, 256)
    pos_sg = pg_f.reshape(32, M, 128)[:, :, :3]

    out = _edge_pipeline(pos_s, pos_sg, a_dst, a_srcg, xval_g, w_edge,
                         W_p1, b_p1, g_p1, be_p1, W_p2, b_p2, g_p2, be_p2,
                         W_a1, b_a1, g_a1, be_a1, W_a2, b_a2, g_a2, be_a2)
    out = _pallas_matmul(out, W_up) + b_up
    return out + x_s
